# initial kernel scaffold (unmeasured)
import jax
import jax.numpy as jnp
from jax import lax
from jax.experimental import pallas as pl
from jax.experimental.pallas import tpu as pltpu

N_DEV = 4
N_LAYERS = 3


def kernel(x, Win0, Wout0, Win1, Wout1, Win2, Wout2):
    b, d = x.shape
    b_out = b // N_DEV

    def body(x_ref, win0_ref, wout0_ref, win1_ref, wout1_ref,
             win2_ref, wout2_ref, out_ref, comm_ref, send_sems, recv_sems):
        my = lax.axis_index("i")

        barrier_sem = pltpu.get_barrier_semaphore()
        for off in range(1, N_DEV):
            pl.semaphore_signal(
                barrier_sem, inc=1,
                device_id=((my + off) % N_DEV,),
                device_id_type=pl.DeviceIdType.MESH,
            )
        pl.semaphore_wait(barrier_sem, N_DEV - 1)

        wins = [win0_ref, win1_ref, win2_ref]
        wouts = [wout0_ref, wout1_ref, wout2_ref]

        xv = x_ref[...]
        for l in range(N_LAYERS):
            h = jnp.dot(xv.astype(jnp.bfloat16), wins[l][...].astype(jnp.bfloat16),
                        preferred_element_type=jnp.float32)
            h = jnp.maximum(h, 0.0)
            partial = jnp.dot(h.astype(jnp.bfloat16),
                              wouts[l][...].astype(jnp.bfloat16),
                              preferred_element_type=jnp.float32)

            comm_ref[l, 0] = partial.astype(jnp.bfloat16)

            sends = []
            for off in range(1, N_DEV):
                j = N_DEV - off
                rdma = pltpu.make_async_remote_copy(
                    src_ref=comm_ref.at[l, 0],
                    dst_ref=comm_ref.at[l, j],
                    send_sem=send_sems.at[l, off],
                    recv_sem=recv_sems.at[l, j],
                    device_id=((my + off) % N_DEV,),
                    device_id_type=pl.DeviceIdType.MESH,
                )
                rdma.start()
                sends.append(rdma)

            for j in range(1, N_DEV):
                recv = pltpu.make_async_remote_copy(
                    src_ref=comm_ref.at[l, 0],
                    dst_ref=comm_ref.at[l, j],
                    send_sem=send_sems.at[l, 0],
                    recv_sem=recv_sems.at[l, j],
                    device_id=(my,),
                    device_id_type=pl.DeviceIdType.MESH,
                )
                recv.wait_recv()
            for rdma in sends:
                rdma.wait_send()

            xv = (comm_ref[l, 0].astype(jnp.float32)
                  + comm_ref[l, 1].astype(jnp.float32)
                  + comm_ref[l, 2].astype(jnp.float32)
                  + comm_ref[l, 3].astype(jnp.float32))

        out_ref[...] = lax.dynamic_slice(xv, (my * b_out, 0), (b_out, d))

    return pl.pallas_call(
        body,
        out_shape=jax.ShapeDtypeStruct((b_out, d), jnp.float32),
        in_specs=[pl.BlockSpec(memory_space=pltpu.VMEM)] * 7,
        out_specs=pl.BlockSpec(memory_space=pltpu.VMEM),
        scratch_shapes=[
            pltpu.VMEM((N_LAYERS, N_DEV, b, d), jnp.bfloat16),
            pltpu.SemaphoreType.DMA((N_LAYERS, N_DEV)),
            pltpu.SemaphoreType.DMA((N_LAYERS, N_DEV)),
        ],
        compiler_params=pltpu.CompilerParams(collective_id=0),
    )(x, Win0, Wout0, Win1, Wout1, Win2, Wout2)


# baseline (device time: 27220 ns/iter reference)
import jax
import jax.numpy as jnp
from jax import lax
from jax.experimental import pallas as pl
from jax.experimental.pallas import tpu as pltpu

N_DEV = 4
N_LAYERS = 3


def kernel(x, Win0, Wout0, Win1, Wout1, Win2, Wout2):
    b, d = x.shape
    b_out = b // N_DEV

    def body(x_ref, win0_ref, wout0_ref, win1_ref, wout1_ref,
             win2_ref, wout2_ref, out_ref, comm_ref, send_sems, recv_sems):
        my = lax.axis_index("i")

        barrier_sem = pltpu.get_barrier_semaphore()
        for off in range(1, N_DEV):
            pl.semaphore_signal(
                barrier_sem, inc=1,
                device_id=((my + off) % N_DEV,),
                device_id_type=pl.DeviceIdType.MESH,
            )
        pl.semaphore_wait(barrier_sem, N_DEV - 1)

        wins = [win0_ref, win1_ref, win2_ref]
        wouts = [wout0_ref, wout1_ref, wout2_ref]

        xv = x_ref[...]
        for l in range(N_LAYERS):
            h = jnp.dot(xv.astype(jnp.bfloat16), wins[l][...].astype(jnp.bfloat16),
                        preferred_element_type=jnp.float32)
            h = jnp.maximum(h, 0.0)
            partial = jnp.dot(h.astype(jnp.bfloat16),
                              wouts[l][...].astype(jnp.bfloat16),
                              preferred_element_type=jnp.float32)

            comm_ref[l, 0] = partial.astype(jnp.bfloat16)

            sends = []
            for off in range(1, N_DEV):
                j = N_DEV - off
                rdma = pltpu.make_async_remote_copy(
                    src_ref=comm_ref.at[l, 0],
                    dst_ref=comm_ref.at[l, j],
                    send_sem=send_sems.at[l, off],
                    recv_sem=recv_sems.at[l, j],
                    device_id=((my + off) % N_DEV,),
                    device_id_type=pl.DeviceIdType.MESH,
                )
                rdma.start()
                sends.append(rdma)

            for j in range(1, N_DEV):
                recv = pltpu.make_async_remote_copy(
                    src_ref=comm_ref.at[l, 0],
                    dst_ref=comm_ref.at[l, j],
                    send_sem=send_sems.at[l, 0],
                    recv_sem=recv_sems.at[l, j],
                    device_id=(my,),
                    device_id_type=pl.DeviceIdType.MESH,
                )
                recv.wait_recv()
            for rdma in sends:
                rdma.wait_send()

            if l < N_LAYERS - 1:
                xv = (comm_ref[l, 0].astype(jnp.float32)
                      + comm_ref[l, 1].astype(jnp.float32)
                      + comm_ref[l, 2].astype(jnp.float32)
                      + comm_ref[l, 3].astype(jnp.float32))

        rows = pl.ds(my * b_out, b_out)
        last = N_LAYERS - 1
        out_ref[...] = (comm_ref[last, 0, rows, :].astype(jnp.float32)
                        + comm_ref[last, 1, rows, :].astype(jnp.float32)
                        + comm_ref[last, 2, rows, :].astype(jnp.float32)
                        + comm_ref[last, 3, rows, :].astype(jnp.float32))

    return pl.pallas_call(
        body,
        out_shape=jax.ShapeDtypeStruct((b_out, d), jnp.float32),
        in_specs=[pl.BlockSpec(memory_space=pltpu.VMEM)] * 7,
        out_specs=pl.BlockSpec(memory_space=pltpu.VMEM),
        scratch_shapes=[
            pltpu.VMEM((N_LAYERS, N_DEV, b, d), jnp.bfloat16),
            pltpu.SemaphoreType.DMA((N_LAYERS, N_DEV)),
            pltpu.SemaphoreType.DMA((N_LAYERS, N_DEV)),
        ],
        compiler_params=pltpu.CompilerParams(collective_id=0),
    )(x, Win0, Wout0, Win1, Wout1, Win2, Wout2)


# device time: 26122 ns/iter; 1.0420x vs baseline; 1.0420x over previous
import jax
import jax.numpy as jnp
from jax import lax
from jax.experimental import pallas as pl
from jax.experimental.pallas import tpu as pltpu

N_DEV = 4
N_LAYERS = 3


def kernel(x, Win0, Wout0, Win1, Wout1, Win2, Wout2):
    b, d = x.shape
    b_out = b // N_DEV

    def body(x_ref, win0_ref, wout0_ref, win1_ref, wout1_ref,
             win2_ref, wout2_ref, out_ref, comm_ref, rs_ref,
             send_sems, recv_sems):
        my = lax.axis_index("i")

        barrier_sem = pltpu.get_barrier_semaphore()
        for off in range(1, N_DEV):
            pl.semaphore_signal(
                barrier_sem, inc=1,
                device_id=((my + off) % N_DEV,),
                device_id_type=pl.DeviceIdType.MESH,
            )
        pl.semaphore_wait(barrier_sem, N_DEV - 1)

        wins = [win0_ref, win1_ref, win2_ref]
        wouts = [wout0_ref, wout1_ref, wout2_ref]

        def mm(a, w):
            return jnp.dot(a, w, preferred_element_type=jnp.float32)

        def broadcast_partial(l, partial_f32):
            comm_ref[l, 0] = partial_f32.astype(jnp.bfloat16)
            sends = []
            for off in range(1, N_DEV):
                j = N_DEV - off
                rdma = pltpu.make_async_remote_copy(
                    src_ref=comm_ref.at[l, 0],
                    dst_ref=comm_ref.at[l, j],
                    send_sem=send_sems.at[l, off],
                    recv_sem=recv_sems.at[l, j],
                    device_id=((my + off) % N_DEV,),
                    device_id_type=pl.DeviceIdType.MESH,
                )
                rdma.start()
                sends.append(rdma)
            return sends

        def wait_slot(l, j):
            recv = pltpu.make_async_remote_copy(
                src_ref=comm_ref.at[l, 0],
                dst_ref=comm_ref.at[l, j],
                send_sem=send_sems.at[l, 0],
                recv_sem=recv_sems.at[l, j],
                device_id=(my,),
                device_id_type=pl.DeviceIdType.MESH,
            )
            recv.wait_recv()

        xb = x_ref[...].astype(jnp.bfloat16)
        h = jnp.maximum(mm(xb, win0_ref[...].astype(jnp.bfloat16)), 0.0)
        partial = mm(h.astype(jnp.bfloat16), wout0_ref[...].astype(jnp.bfloat16))
        sends = broadcast_partial(0, partial)

        for l in range(1, N_LAYERS):
            win_bf = wins[l][...].astype(jnp.bfloat16)
            h = mm(comm_ref[l - 1, 0], win_bf)
            for j in range(1, N_DEV):
                wait_slot(l - 1, j)
                h = h + mm(comm_ref[l - 1, j], win_bf)
            for rdma in sends:
                rdma.wait_send()
            h = jnp.maximum(h, 0.0)
            partial = mm(h.astype(jnp.bfloat16),
                         wouts[l][...].astype(jnp.bfloat16))

            if l < N_LAYERS - 1:
                sends = broadcast_partial(l, partial)
            else:
                comm_ref[l, 0] = partial.astype(jnp.bfloat16)
                sends = []
                for off in range(1, N_DEV):
                    j = N_DEV - off
                    peer = (my + off) % N_DEV
                    rdma = pltpu.make_async_remote_copy(
                        src_ref=comm_ref.at[l, 0, pl.ds(peer * b_out, b_out)],
                        dst_ref=rs_ref.at[j],
                        send_sem=send_sems.at[l, off],
                        recv_sem=recv_sems.at[l, j],
                        device_id=(peer,),
                        device_id_type=pl.DeviceIdType.MESH,
                    )
                    rdma.start()
                    sends.append(rdma)

        last = N_LAYERS - 1
        for j in range(1, N_DEV):
            recv = pltpu.make_async_remote_copy(
                src_ref=comm_ref.at[last, 0, pl.ds(0, b_out)],
                dst_ref=rs_ref.at[j],
                send_sem=send_sems.at[last, 0],
                recv_sem=recv_sems.at[last, j],
                device_id=(my,),
                device_id_type=pl.DeviceIdType.MESH,
            )
            recv.wait_recv()
        for rdma in sends:
            rdma.wait_send()

        rows = pl.ds(my * b_out, b_out)
        out_ref[...] = (comm_ref[last, 0, rows, :].astype(jnp.float32)
                        + rs_ref[1].astype(jnp.float32)
                        + rs_ref[2].astype(jnp.float32)
                        + rs_ref[3].astype(jnp.float32))

    return pl.pallas_call(
        body,
        out_shape=jax.ShapeDtypeStruct((b_out, d), jnp.float32),
        in_specs=[pl.BlockSpec(memory_space=pltpu.VMEM)] * 7,
        out_specs=pl.BlockSpec(memory_space=pltpu.VMEM),
        scratch_shapes=[
            pltpu.VMEM((N_LAYERS, N_DEV, b, d), jnp.bfloat16),
            pltpu.VMEM((N_DEV, b_out, d), jnp.bfloat16),
            pltpu.SemaphoreType.DMA((N_LAYERS, N_DEV)),
            pltpu.SemaphoreType.DMA((N_LAYERS, N_DEV)),
        ],
        compiler_params=pltpu.CompilerParams(collective_id=0),
    )(x, Win0, Wout0, Win1, Wout1, Win2, Wout2)


# device time: 24237 ns/iter; 1.1231x vs baseline; 1.0778x over previous
import jax
import jax.numpy as jnp
from jax import lax
from jax.experimental import pallas as pl
from jax.experimental.pallas import tpu as pltpu

N_DEV = 4
N_LAYERS = 3


def kernel(x, Win0, Wout0, Win1, Wout1, Win2, Wout2):
    b, d = x.shape
    b_out = b // N_DEV

    def body(x_ref, win0_ref, wout0_ref, win1_ref, wout1_ref,
             win2_ref, wout2_ref, out_ref, comm_ref, rs_ref,
             send_sems, recv_sems):
        my = lax.axis_index("i")
        bf = jnp.bfloat16

        barrier_sem = pltpu.get_barrier_semaphore()
        for off in range(1, N_DEV):
            pl.semaphore_signal(
                barrier_sem, inc=1,
                device_id=((my + off) % N_DEV,),
                device_id_type=pl.DeviceIdType.MESH,
            )

        xb = x_ref[...].astype(bf)
        win_bf = win0_ref[...].astype(bf)
        wout_bf = wout0_ref[...].astype(bf)

        wins = [win0_ref, win1_ref, win2_ref]
        wouts = [wout0_ref, wout1_ref, wout2_ref]

        def mm(a, w):
            return jnp.dot(a, w, preferred_element_type=jnp.float32)

        def recv_desc(l, j, dst):
            return pltpu.make_async_remote_copy(
                src_ref=comm_ref.at[l, 0, pl.ds(0, dst.shape[0])],
                dst_ref=dst,
                send_sem=send_sems.at[l, 0],
                recv_sem=recv_sems.at[l, j],
                device_id=(my,),
                device_id_type=pl.DeviceIdType.MESH,
            )

        all_sends = []
        for l in range(N_LAYERS):
            h = jnp.maximum(mm(xb, win_bf), 0.0)
            partial = mm(h.astype(bf), wout_bf)
            comm_ref[l, 0] = partial.astype(bf)

            if l == 0:
                pl.semaphore_wait(barrier_sem, N_DEV - 1)

            last = l == N_LAYERS - 1
            for off in (2, 1, 3):
                j = N_DEV - off
                peer = (my + off) % N_DEV
                if last:
                    src = comm_ref.at[l, 0, pl.ds(peer * b_out, b_out)]
                    dst = rs_ref.at[j]
                else:
                    src = comm_ref.at[l, 0]
                    dst = comm_ref.at[l, j]
                rdma = pltpu.make_async_remote_copy(
                    src_ref=src,
                    dst_ref=dst,
                    send_sem=send_sems.at[l, off],
                    recv_sem=recv_sems.at[l, j],
                    device_id=(peer,),
                    device_id_type=pl.DeviceIdType.MESH,
                )
                rdma.start()
                all_sends.append(rdma)

            if last:
                break

            win_bf = wins[l + 1][...].astype(bf)
            wout_bf = wouts[l + 1][...].astype(bf)

            recv_desc(l, 1, comm_ref.at[l, 1]).wait_recv()
            recv_desc(l, 3, comm_ref.at[l, 3]).wait_recv()
            acc = (comm_ref[l, 0].astype(jnp.float32)
                   + comm_ref[l, 1].astype(jnp.float32)
                   + comm_ref[l, 3].astype(jnp.float32))
            recv_desc(l, 2, comm_ref.at[l, 2]).wait_recv()
            xb = (acc + comm_ref[l, 2].astype(jnp.float32)).astype(bf)

        last = N_LAYERS - 1
        rows = pl.ds(my * b_out, b_out)
        recv_desc(last, 1, rs_ref.at[1]).wait_recv()
        recv_desc(last, 3, rs_ref.at[3]).wait_recv()
        acc = (comm_ref[last, 0, rows, :].astype(jnp.float32)
               + rs_ref[1].astype(jnp.float32)
               + rs_ref[3].astype(jnp.float32))
        recv_desc(last, 2, rs_ref.at[2]).wait_recv()
        out_ref[...] = acc + rs_ref[2].astype(jnp.float32)

        for rdma in all_sends:
            rdma.wait_send()

    return pl.pallas_call(
        body,
        out_shape=jax.ShapeDtypeStruct((b_out, d), jnp.float32),
        in_specs=[pl.BlockSpec(memory_space=pltpu.VMEM)] * 7,
        out_specs=pl.BlockSpec(memory_space=pltpu.VMEM),
        scratch_shapes=[
            pltpu.VMEM((N_LAYERS, N_DEV, b, d), jnp.bfloat16),
            pltpu.VMEM((N_DEV, b_out, d), jnp.bfloat16),
            pltpu.SemaphoreType.DMA((N_LAYERS, N_DEV)),
            pltpu.SemaphoreType.DMA((N_LAYERS, N_DEV)),
        ],
        compiler_params=pltpu.CompilerParams(collective_id=0),
    )(x, Win0, Wout0, Win1, Wout1, Win2, Wout2)
